# baseline (speedup 1.0000x reference)
import functools
import jax
import jax.numpy as jnp
from jax import lax
from jax.experimental import pallas as pl
from jax.experimental.pallas import tpu as pltpu
from jax.experimental.pallas import tpu_sc as plsc

N = 10000
E = 320000
DIN = 128
DH = 256
DC = 4
EPS = 1e-5

NC = 2
NS = 16
NW = NC * NS

CHUNK = 80
NP = 10240
RPS = NP // NS
QD = NP // 128

_mesh = plsc.VectorSubcoreMesh(core_axis_name="c", subcore_axis_name="s")


EP = 327680
DEG_ROWS = 32


def _deg_tc(dst_r, deg_r, rdeg_r, mdeg_r):
    i = pl.program_id(0)

    @pl.when(i == 0)
    def _():
        deg_r[...] = jnp.zeros_like(deg_r)

    d = dst_r[...]
    q = d // 128
    r = d % 128
    u = (r[:, :, None] == lax.broadcasted_iota(
        jnp.int32, (DEG_ROWS, 128, 128), 2)).astype(jnp.bfloat16)
    v = (q[:, :, None] == lax.broadcasted_iota(
        jnp.int32, (DEG_ROWS, 128, QD), 2)).astype(jnp.bfloat16)
    u2 = u.reshape(DEG_ROWS * 128, 128)
    v2 = v.reshape(DEG_ROWS * 128, QD)
    deg_r[...] += lax.dot_general(
        v2, u2, (((0,), (0,)), ((), ())),
        preferred_element_type=jnp.float32)

    @pl.when(i == pl.num_programs(0) - 1)
    def _():
        rdeg_r[...] = 1.0 / jnp.maximum(deg_r[...], 1.0)
        mdeg_r[...] = jnp.max(deg_r[...]).reshape(1, 1)


def _degree(dst2d):
    grid = ((EP // 128) // DEG_ROWS,)
    return pl.pallas_call(
        _deg_tc,
        grid=grid,
        in_specs=[pl.BlockSpec((DEG_ROWS, 128), lambda i: (i, 0))],
        out_specs=[pl.BlockSpec((QD, 128), lambda i: (0, 0)),
                   pl.BlockSpec((QD, 128), lambda i: (0, 0)),
                   pl.BlockSpec((1, 1), lambda i: (0, 0))],
        out_shape=[jax.ShapeDtypeStruct((QD, 128), jnp.float32),
                   jax.ShapeDtypeStruct((QD, 128), jnp.float32),
                   jax.ShapeDtypeStruct((1, 1), jnp.float32)],
    )(dst2d)


NBUF = 4


def _make_agg(row_shape, dtype):
    scratch = [
        pltpu.VMEM((3, NBUF, CHUNK), jnp.int32),
        pltpu.VMEM((3, NBUF, CHUNK), jnp.int32),
        pltpu.VMEM((NBUF, CHUNK) + row_shape, dtype),
        pltpu.VMEM_SHARED((NP,) + row_shape, dtype),
    ] + [pltpu.SemaphoreType.DMA] * (NBUF + 2)

    @functools.partial(
        pl.kernel,
        out_type=jax.ShapeDtypeStruct((NC, NP) + row_shape, dtype),
        mesh=_mesh,
        scratch_types=scratch,
    )
    def k(table, src2, dst2, zinit, out, src_i, dst_i, rows, acc, *sems):
        gsem = sems[:NBUF]
        ssem = sems[NBUF]
        isem = sems[NBUF + 1]
        c = lax.axis_index("c")
        s = lax.axis_index("s")
        pltpu.sync_copy(zinit.at[pl.ds(s * RPS, RPS)],
                        acc.at[pl.ds(s * RPS, RPS)])
        plsc.subcore_barrier()

        tblk = E // (CHUNK * NBUF)
        per, extra = tblk // NW, tblk % NW
        wid = s * NC + c
        tab = table
        nblk = per + jnp.where(wid < extra, 1, 0)
        base = wid * per + jnp.minimum(wid, extra)

        pltpu.async_copy(src2.at[base], src_i.at[0], isem)
        pltpu.async_copy(dst2.at[base], dst_i.at[0], isem)

        def outer(g, carry):
            p = lax.rem(g, 3)
            pltpu.make_async_copy(src2.at[base], src_i.at[p], isem).wait()
            pltpu.make_async_copy(dst2.at[base], dst_i.at[p], isem).wait()

            @pl.when(g + 1 < nblk)
            def _():
                pn = lax.rem(g + 1, 3)
                pltpu.async_copy(src2.at[base + g + 1], src_i.at[pn], isem)
                pltpu.async_copy(dst2.at[base + g + 1], dst_i.at[pn], isem)

            @pl.when(g > 0)
            def _():
                for b in range(NBUF):
                    pltpu.make_async_copy(zinit.at[pl.ds(0, CHUNK)],
                                          rows.at[b], ssem).wait()

            gs = [pltpu.async_copy(tab.at[src_i.at[p].at[b]], rows.at[b],
                                   gsem[b]) for b in range(NBUF)]
            for b in range(NBUF):
                gs[b].wait()
                pltpu.async_copy(rows.at[b], acc.at[dst_i.at[p].at[b]],
                                 ssem, add=True)
            return carry

        lax.fori_loop(0, nblk, outer, 0)
        for b in range(NBUF):
            pltpu.make_async_copy(zinit.at[pl.ds(0, CHUNK)], rows.at[b],
                                  ssem).wait()
        plsc.subcore_barrier()
        pltpu.sync_copy(acc.at[pl.ds(s * RPS, RPS)],
                        out.at[c, pl.ds(s * RPS, RPS)])

    return k


_agg1_sc = _make_agg((DIN,), jnp.float32)
_agg2_sc = _make_agg((DIN,), jnp.int32)


BLK = 1000


def _mm_tc(a, W, b, o):
    o[...] = (jnp.dot(a[...], W[...], preferred_element_type=jnp.float32)
              + b[...])


def _mm(a, W, b):
    K = a.shape[1]
    return pl.pallas_call(
        _mm_tc,
        grid=(N // BLK,),
        in_specs=[pl.BlockSpec((BLK, K), lambda i: (i, 0)),
                  pl.BlockSpec((K, DH), lambda i: (0, 0)),
                  pl.BlockSpec((1, DH), lambda i: (0, 0))],
        out_specs=pl.BlockSpec((BLK, DH), lambda i: (i, 0)),
        out_shape=jax.ShapeDtypeStruct((N, DH), jnp.float32),
    )(a, W, b)


def _mmh_tc(h2, W, b, o):
    h = jnp.concatenate([h2[0], h2[1]], axis=1)
    o[...] = (jnp.dot(h, W[...], preferred_element_type=jnp.float32)
              + b[...])


def _mmh(h2, W, b):
    return pl.pallas_call(
        _mmh_tc,
        grid=(N // BLK,),
        in_specs=[pl.BlockSpec((NC, BLK, DIN), lambda i: (0, i, 0)),
                  pl.BlockSpec((DH, DH), lambda i: (0, 0)),
                  pl.BlockSpec((1, DH), lambda i: (0, 0))],
        out_specs=pl.BlockSpec((BLK, DH), lambda i: (i, 0)),
        out_shape=jax.ShapeDtypeStruct((N, DH), jnp.float32),
    )(h2, W, b)


def _layer1_tc(aggp, xw, rdeg, Wl, g, be, rm, rv, h2, mh):
    i = pl.program_id(0)
    mean = (aggp[0] + aggp[1]) * rdeg[...]
    t = (jnp.dot(mean, Wl[...], preferred_element_type=jnp.float32)
         + xw[...])
    t = (t - rm[...]) / jnp.sqrt(rv[...] + EPS) * g[...] + be[...]
    h = jnp.maximum(t, 0.0)
    h2[0] = h[:, :DIN]
    h2[1] = h[:, DIN:]

    @pl.when(i == 0)
    def _():
        mh[...] = jnp.zeros_like(mh)

    mh[...] = jnp.maximum(mh[...], jnp.max(h))


def _phase_b(aggp, xw, rdeg, Wl, g, be, rm, rv):
    grid = (N // BLK,)
    pspec = pl.BlockSpec((1, DH), lambda i: (0, 0))
    return pl.pallas_call(
        _layer1_tc,
        grid=grid,
        in_specs=[
            pl.BlockSpec((NC, BLK, DIN), lambda i: (0, i, 0)),
            pl.BlockSpec((BLK, DH), lambda i: (i, 0)),
            pl.BlockSpec((BLK, 1), lambda i: (i, 0)),
            pl.BlockSpec((DIN, DH), lambda i: (0, 0)),
            pspec, pspec, pspec, pspec,
        ],
        out_specs=[pl.BlockSpec((NC, BLK, DIN), lambda i: (0, i, 0)),
                   pl.BlockSpec((1, 1), lambda i: (0, 0))],
        out_shape=[jax.ShapeDtypeStruct((NC, N, DIN), jnp.float32),
                   jax.ShapeDtypeStruct((1, 1), jnp.float32)],
    )(aggp, xw, rdeg, Wl, g, be, rm, rv)


def _quant_tc(h2, mh, mdeg, hq, sc_r):
    md = mdeg[...]
    num = jnp.maximum(64000.0 - 0.5 * md, 1.0)
    den = md * mh[...]
    scale = jnp.where(den > 0, num / jnp.maximum(den, 1e-30), 1.0)
    sc_r[...] = scale
    q0 = jnp.round(h2[0] * scale).astype(jnp.int32)
    q1 = jnp.round(h2[1] * scale).astype(jnp.int32)
    hq[...] = q0 + lax.shift_left(q1, 16)


def _quant(h2, mh, mdeg):
    grid = (N // BLK,)
    sspec = pl.BlockSpec((1, 1), lambda i: (0, 0))
    return pl.pallas_call(
        _quant_tc,
        grid=grid,
        in_specs=[pl.BlockSpec((NC, BLK, DIN), lambda i: (0, i, 0)),
                  sspec, sspec],
        out_specs=[pl.BlockSpec((BLK, DIN), lambda i: (i, 0)),
                   sspec],
        out_shape=[jax.ShapeDtypeStruct((N, DIN), jnp.int32),
                   jax.ShapeDtypeStruct((1, 1), jnp.float32)],
    )(h2, mh, mdeg)


def _layer2_tc(agg2, hw, rdeg, scale, Wl, g, be, rm, rv, Wc1, bc1,
               Wc2, bc2, emb_r, log_r):
    w = agg2[0] + agg2[1]
    lo = (w & 0xFFFF).astype(jnp.float32)
    hi = lax.shift_right_logical(w, 16).astype(jnp.float32)
    a = jnp.concatenate([lo, hi], axis=1)
    mean = a * (rdeg[...] / scale[...])
    t = (jnp.dot(mean, Wl[...], preferred_element_type=jnp.float32)
         + hw[...])
    t = (t - rm[...]) / jnp.sqrt(rv[...] + EPS) * g[...] + be[...]
    emb = jnp.maximum(t, 0.0)
    hc = jnp.maximum(
        jnp.dot(emb, Wc1[...], preferred_element_type=jnp.float32) + bc1[...],
        0.0)
    logits = jnp.dot(hc, Wc2[...], preferred_element_type=jnp.float32) + bc2[...]
    emb_r[...] = emb
    log_r[...] = logits


def _phase_d(agg2, hw, rdeg, scale, Wl, g, be, rm, rv, Wc1, bc1,
             Wc2p, bc2p):
    grid = (N // BLK,)
    pspec = pl.BlockSpec((1, DH), lambda i: (0, 0))
    return pl.pallas_call(
        _layer2_tc,
        grid=grid,
        in_specs=[
            pl.BlockSpec((NC, BLK, DIN), lambda i: (0, i, 0)),
            pl.BlockSpec((BLK, DH), lambda i: (i, 0)),
            pl.BlockSpec((BLK, 1), lambda i: (i, 0)),
            pl.BlockSpec((1, 1), lambda i: (0, 0)),
            pl.BlockSpec((DH, DH), lambda i: (0, 0)),
            pspec, pspec, pspec, pspec,
            pl.BlockSpec((DH, DH // 2), lambda i: (0, 0)),
            pl.BlockSpec((1, DH // 2), lambda i: (0, 0)),
            pl.BlockSpec((DH // 2, 128), lambda i: (0, 0)),
            pl.BlockSpec((1, 128), lambda i: (0, 0)),
        ],
        out_specs=[
            pl.BlockSpec((BLK, DH), lambda i: (i, 0)),
            pl.BlockSpec((BLK, 128), lambda i: (i, 0)),
        ],
        out_shape=[
            jax.ShapeDtypeStruct((N, DH), jnp.float32),
            jax.ShapeDtypeStruct((N, 128), jnp.float32),
        ],
    )(agg2, hw, rdeg, scale, Wl, g, be, rm, rv, Wc1, bc1, Wc2p, bc2p)


def kernel(x, edge_index, Wl1, Wr1, b1, g1, be1, rm1, rv1,
           Wl2, Wr2, b2, g2, be2, rm2, rv2, Wc1, bc1, Wc2, bc2):
    src = edge_index[0]
    dst = edge_index[1]
    src2 = src.reshape(E // (NBUF * CHUNK), NBUF, CHUNK)
    dst2 = dst.reshape(E // (NBUF * CHUNK), NBUF, CHUNK)
    zA = jnp.zeros((NP, DIN), jnp.float32)
    zA32 = jnp.zeros((NP, DIN), jnp.int32)

    row = lambda v: v.reshape(1, -1)

    aggp = _agg1_sc(x, src2, dst2, zA)[:, :N]
    dstp = jnp.concatenate([dst, jnp.full((EP - E,), NP, jnp.int32)])
    _, rmat, mdeg = _degree(dstp.reshape(EP // 128, 128))
    rdeg = rmat.reshape(NP, 1)[:N]
    xw = _mm(x, Wr1, row(b1))

    h2, mh = _phase_b(aggp, xw, rdeg, Wl1, row(g1), row(be1),
                      row(rm1), row(rv1))

    hq, scale = _quant(h2, mh, mdeg)
    agg2 = _agg2_sc(hq, src2, dst2, zA32)[:, :N]
    hw = _mmh(h2, Wr2, row(b2))

    Wc2p = jnp.pad(Wc2, ((0, 0), (0, 128 - DC)))
    bc2p = jnp.pad(bc2, (0, 128 - DC)).reshape(1, -1)
    emb, logp = _phase_d(agg2, hw, rdeg, scale, Wl2, row(g2),
                         row(be2), row(rm2), row(rv2), Wc1, row(bc1),
                         Wc2p, bc2p)
    return (logp[:, :DC], emb)


# speedup vs baseline: 1.0063x; 1.0063x over previous
import functools
import jax
import jax.numpy as jnp
from jax import lax
from jax.experimental import pallas as pl
from jax.experimental.pallas import tpu as pltpu
from jax.experimental.pallas import tpu_sc as plsc

N = 10000
E = 320000
DIN = 128
DH = 256
DC = 4
EPS = 1e-5

NC = 2
NS = 16
NW = NC * NS

CHUNK = 80
NP = 10240
RPS = NP // NS
QD = NP // 128

_mesh = plsc.VectorSubcoreMesh(core_axis_name="c", subcore_axis_name="s")


EP = 327680
DEG_ROWS = 32


def _deg_tc(dst_r, deg_r, rdeg_r, mdeg_r):
    i = pl.program_id(0)

    @pl.when(i == 0)
    def _():
        deg_r[...] = jnp.zeros_like(deg_r)

    d = dst_r[...]
    q = d // 128
    r = d % 128
    u = (r[:, :, None] == lax.broadcasted_iota(
        jnp.int32, (DEG_ROWS, 128, 128), 2)).astype(jnp.bfloat16)
    v = (q[:, :, None] == lax.broadcasted_iota(
        jnp.int32, (DEG_ROWS, 128, QD), 2)).astype(jnp.bfloat16)
    u2 = u.reshape(DEG_ROWS * 128, 128)
    v2 = v.reshape(DEG_ROWS * 128, QD)
    deg_r[...] += lax.dot_general(
        v2, u2, (((0,), (0,)), ((), ())),
        preferred_element_type=jnp.float32)

    @pl.when(i == pl.num_programs(0) - 1)
    def _():
        rdeg_r[...] = 1.0 / jnp.maximum(deg_r[...], 1.0)
        mdeg_r[...] = jnp.max(deg_r[...]).reshape(1, 1)


def _degree(dst2d):
    grid = ((EP // 128) // DEG_ROWS,)
    return pl.pallas_call(
        _deg_tc,
        grid=grid,
        in_specs=[pl.BlockSpec((DEG_ROWS, 128), lambda i: (i, 0))],
        out_specs=[pl.BlockSpec((QD, 128), lambda i: (0, 0)),
                   pl.BlockSpec((QD, 128), lambda i: (0, 0)),
                   pl.BlockSpec((1, 1), lambda i: (0, 0))],
        out_shape=[jax.ShapeDtypeStruct((QD, 128), jnp.float32),
                   jax.ShapeDtypeStruct((QD, 128), jnp.float32),
                   jax.ShapeDtypeStruct((1, 1), jnp.float32)],
    )(dst2d)


NBUF = 4


def _make_agg(row_shape, dtype):
    scratch = [
        pltpu.VMEM((3, NBUF, CHUNK), jnp.int32),
        pltpu.VMEM((3, NBUF, CHUNK), jnp.int32),
        pltpu.VMEM((NBUF, CHUNK) + row_shape, dtype),
        pltpu.VMEM_SHARED((NP,) + row_shape, dtype),
    ] + [pltpu.SemaphoreType.DMA] * (NBUF + 2)

    @functools.partial(
        pl.kernel,
        out_type=jax.ShapeDtypeStruct((NC, NP) + row_shape, dtype),
        mesh=_mesh,
        scratch_types=scratch,
    )
    def k(table, src2, dst2, zinit, out, src_i, dst_i, rows, acc, *sems):
        gsem = sems[:NBUF]
        ssem = sems[NBUF]
        isem = sems[NBUF + 1]
        c = lax.axis_index("c")
        s = lax.axis_index("s")
        pltpu.sync_copy(zinit.at[pl.ds(s * RPS, RPS)],
                        acc.at[pl.ds(s * RPS, RPS)])
        plsc.subcore_barrier()

        tblk = E // (CHUNK * NBUF)
        per, extra = tblk // NW, tblk % NW
        wid = s * NC + c
        tab = table
        nblk = per + jnp.where(wid < extra, 1, 0)
        base = wid * per + jnp.minimum(wid, extra)

        pltpu.async_copy(src2.at[base], src_i.at[0], isem)
        pltpu.async_copy(dst2.at[base], dst_i.at[0], isem)

        def outer(g, carry):
            p = lax.rem(g, 3)
            pltpu.make_async_copy(src2.at[base], src_i.at[p], isem).wait()
            pltpu.make_async_copy(dst2.at[base], dst_i.at[p], isem).wait()

            @pl.when(g + 1 < nblk)
            def _():
                pn = lax.rem(g + 1, 3)
                pltpu.async_copy(src2.at[base + g + 1], src_i.at[pn], isem)
                pltpu.async_copy(dst2.at[base + g + 1], dst_i.at[pn], isem)

            @pl.when(g > 0)
            def _():
                for b in range(NBUF):
                    pltpu.make_async_copy(zinit.at[pl.ds(0, CHUNK)],
                                          rows.at[b], ssem).wait()

            gs = [pltpu.async_copy(tab.at[src_i.at[p].at[b]], rows.at[b],
                                   gsem[b]) for b in range(NBUF)]
            for b in range(NBUF):
                gs[b].wait()
                pltpu.async_copy(rows.at[b], acc.at[dst_i.at[p].at[b]],
                                 ssem, add=True)
            return carry

        lax.fori_loop(0, nblk, outer, 0)
        for b in range(NBUF):
            pltpu.make_async_copy(zinit.at[pl.ds(0, CHUNK)], rows.at[b],
                                  ssem).wait()
        plsc.subcore_barrier()
        pltpu.sync_copy(acc.at[pl.ds(s * RPS, RPS)],
                        out.at[c, pl.ds(s * RPS, RPS)])

    return k


_agg1_sc = _make_agg((DIN,), jnp.float32)
_agg2_sc = _make_agg((DIN,), jnp.int32)


BLK = 1000


def _layer1_tc(aggp, x, rdeg, Wl, Wr, b, g, be, rm, rv, h2, mh):
    i = pl.program_id(0)
    mean = (aggp[0] + aggp[1]) * rdeg[...]
    t = (jnp.dot(mean, Wl[...], preferred_element_type=jnp.float32)
         + jnp.dot(x[...], Wr[...], preferred_element_type=jnp.float32)
         + b[...])
    t = (t - rm[...]) / jnp.sqrt(rv[...] + EPS) * g[...] + be[...]
    h = jnp.maximum(t, 0.0)
    h2[0] = h[:, :DIN]
    h2[1] = h[:, DIN:]

    @pl.when(i == 0)
    def _():
        mh[...] = jnp.zeros_like(mh)

    mh[...] = jnp.maximum(mh[...], jnp.max(h))


def _phase_b(aggp, x, rdeg, Wl, Wr, b, g, be, rm, rv):
    grid = (N // BLK,)
    wspec = pl.BlockSpec((DIN, DH), lambda i: (0, 0))
    pspec = pl.BlockSpec((1, DH), lambda i: (0, 0))
    return pl.pallas_call(
        _layer1_tc,
        grid=grid,
        in_specs=[
            pl.BlockSpec((NC, BLK, DIN), lambda i: (0, i, 0)),
            pl.BlockSpec((BLK, DIN), lambda i: (i, 0)),
            pl.BlockSpec((BLK, 1), lambda i: (i, 0)),
            wspec, wspec, pspec, pspec, pspec, pspec, pspec,
        ],
        out_specs=[pl.BlockSpec((NC, BLK, DIN), lambda i: (0, i, 0)),
                   pl.BlockSpec((1, 1), lambda i: (0, 0))],
        out_shape=[jax.ShapeDtypeStruct((NC, N, DIN), jnp.float32),
                   jax.ShapeDtypeStruct((1, 1), jnp.float32)],
    )(aggp, x, rdeg, Wl, Wr, b, g, be, rm, rv)


def _quant_tc(h2, mh, mdeg, hq, sc_r):
    md = mdeg[...]
    num = jnp.maximum(64000.0 - 0.5 * md, 1.0)
    den = md * mh[...]
    scale = jnp.where(den > 0, num / jnp.maximum(den, 1e-30), 1.0)
    sc_r[...] = scale
    q0 = jnp.round(h2[0] * scale).astype(jnp.int32)
    q1 = jnp.round(h2[1] * scale).astype(jnp.int32)
    hq[...] = q0 + lax.shift_left(q1, 16)


def _quant(h2, mh, mdeg):
    grid = (N // BLK,)
    sspec = pl.BlockSpec((1, 1), lambda i: (0, 0))
    return pl.pallas_call(
        _quant_tc,
        grid=grid,
        in_specs=[pl.BlockSpec((NC, BLK, DIN), lambda i: (0, i, 0)),
                  sspec, sspec],
        out_specs=[pl.BlockSpec((BLK, DIN), lambda i: (i, 0)),
                   sspec],
        out_shape=[jax.ShapeDtypeStruct((N, DIN), jnp.int32),
                   jax.ShapeDtypeStruct((1, 1), jnp.float32)],
    )(h2, mh, mdeg)


def _layer2_tc(agg2, h2, rdeg, scale, Wl, Wr, b, g, be, rm, rv, Wc1, bc1,
               Wc2, bc2, emb_r, log_r):
    w = agg2[0] + agg2[1]
    lo = (w & 0xFFFF).astype(jnp.float32)
    hi = lax.shift_right_logical(w, 16).astype(jnp.float32)
    a = jnp.concatenate([lo, hi], axis=1)
    h = jnp.concatenate([h2[0], h2[1]], axis=1)
    mean = a * (rdeg[...] / scale[...])
    t = (jnp.dot(mean, Wl[...], preferred_element_type=jnp.float32)
         + jnp.dot(h, Wr[...], preferred_element_type=jnp.float32)
         + b[...])
    t = (t - rm[...]) / jnp.sqrt(rv[...] + EPS) * g[...] + be[...]
    emb = jnp.maximum(t, 0.0)
    hc = jnp.maximum(
        jnp.dot(emb, Wc1[...], preferred_element_type=jnp.float32) + bc1[...],
        0.0)
    logits = jnp.dot(hc, Wc2[...], preferred_element_type=jnp.float32) + bc2[...]
    emb_r[...] = emb
    log_r[...] = logits


def _phase_d(agg2, h2, rdeg, scale, Wl, Wr, b, g, be, rm, rv, Wc1, bc1,
             Wc2p, bc2p):
    grid = (N // BLK,)
    w2spec = pl.BlockSpec((DH, DH), lambda i: (0, 0))
    pspec = pl.BlockSpec((1, DH), lambda i: (0, 0))
    return pl.pallas_call(
        _layer2_tc,
        grid=grid,
        in_specs=[
            pl.BlockSpec((NC, BLK, DIN), lambda i: (0, i, 0)),
            pl.BlockSpec((NC, BLK, DIN), lambda i: (0, i, 0)),
            pl.BlockSpec((BLK, 1), lambda i: (i, 0)),
            pl.BlockSpec((1, 1), lambda i: (0, 0)),
            w2spec, w2spec, pspec, pspec, pspec, pspec, pspec,
            pl.BlockSpec((DH, DH // 2), lambda i: (0, 0)),
            pl.BlockSpec((1, DH // 2), lambda i: (0, 0)),
            pl.BlockSpec((DH // 2, 128), lambda i: (0, 0)),
            pl.BlockSpec((1, 128), lambda i: (0, 0)),
        ],
        out_specs=[
            pl.BlockSpec((BLK, DH), lambda i: (i, 0)),
            pl.BlockSpec((BLK, 128), lambda i: (i, 0)),
        ],
        out_shape=[
            jax.ShapeDtypeStruct((N, DH), jnp.float32),
            jax.ShapeDtypeStruct((N, 128), jnp.float32),
        ],
    )(agg2, h2, rdeg, scale, Wl, Wr, b, g, be, rm, rv, Wc1, bc1, Wc2p, bc2p)


def kernel(x, edge_index, Wl1, Wr1, b1, g1, be1, rm1, rv1,
           Wl2, Wr2, b2, g2, be2, rm2, rv2, Wc1, bc1, Wc2, bc2):
    src = edge_index[0]
    dst = edge_index[1]
    src2 = src.reshape(E // (NBUF * CHUNK), NBUF, CHUNK)
    dst2 = dst.reshape(E // (NBUF * CHUNK), NBUF, CHUNK)
    zA = jnp.zeros((NP, DIN), jnp.float32)
    zA32 = jnp.zeros((NP, DIN), jnp.int32)

    dstp = jnp.concatenate([dst, jnp.full((EP - E,), NP, jnp.int32)])
    _, rmat, mdeg = _degree(dstp.reshape(EP // 128, 128))
    rdeg = rmat.reshape(NP, 1)[:N]

    aggp = _agg1_sc(x, src2, dst2, zA)[:, :N]

    row = lambda v: v.reshape(1, -1)
    h2, mh = _phase_b(aggp, x, rdeg, Wl1, Wr1, row(b1), row(g1), row(be1),
                      row(rm1), row(rv1))

    hq, scale = _quant(h2, mh, mdeg)
    agg2 = _agg2_sc(hq, src2, dst2, zA32)[:, :N]

    Wc2p = jnp.pad(Wc2, ((0, 0), (0, 128 - DC)))
    bc2p = jnp.pad(bc2, (0, 128 - DC)).reshape(1, -1)
    emb, logp = _phase_d(agg2, h2, rdeg, scale, Wl2, Wr2, row(b2), row(g2),
                         row(be2), row(rm2), row(rv2), Wc1, row(bc1),
                         Wc2p, bc2p)
    return (logp[:, :DC], emb)
